# baseline (speedup 1.0000x reference)
import jax
import jax.numpy as jnp
from jax import lax
from jax.experimental import pallas as pl
from jax.experimental.pallas import tpu as pltpu
from jax.experimental.pallas import tpu_sc as plsc

_B = 16384
_D = 64
_LANES = 16
_CHUNK = 128

_NC = 2
_NS = 16
_NW = _NC * _NS
_BPW = _B // _NW
_NCHUNK = _BPW // _CHUNK
_NHALF = 2
_HBPW = _BPW // _NHALF
_HCHUNK = _HBPW // _CHUNK
_NGROUP = _HBPW // _LANES

_PANEL = 512
_HU = 524288
_HI = 65536


def _tpose_body(a_ref, b_ref, o_ref):
    r = lax.broadcasted_iota(jnp.int32, (_D, _D), 0)
    c = lax.broadcasted_iota(jnp.int32, (_D, _D), 1)
    eye = (r == c).astype(jnp.float32)
    dn = (((0,), (0,)), ((), ()))
    o_ref[:, 0:_D] = lax.dot_general(
        a_ref[...], eye, dn, preferred_element_type=jnp.float32)
    o_ref[:, _D:2 * _D] = lax.dot_general(
        b_ref[...], eye, dn, preferred_element_type=jnp.float32)


def _pack_halves(tT, half):
    grid = half // _PANEL
    n = tT.shape[1]
    last_b = (n - 1) // _PANEL

    def b_map(i):
        return 0, jnp.minimum(i + half // _PANEL, last_b)

    return pl.pallas_call(
        _tpose_body,
        grid=(grid,),
        in_specs=[
            pl.BlockSpec((_D, _PANEL), lambda i: (0, i)),
            pl.BlockSpec((_D, _PANEL), b_map),
        ],
        out_specs=pl.BlockSpec((_PANEL, 2 * _D), lambda i: (i, 0)),
        out_shape=jax.ShapeDtypeStruct((half, 2 * _D), jnp.float32),
    )(tT, tT)


def _emb_body(uid_hbm, iid_hbm, ut_hbm, it_hbm, dot_hbm,
              uflat_v, iflat_v, uh_v, ih_v, urows_v, irows_v, dot_v, sem):
    wid = lax.axis_index("s") * _NC + lax.axis_index("c")
    base = wid * _BPW

    pltpu.sync_copy(uid_hbm.at[pl.ds(base, _BPW)], uflat_v)
    pltpu.sync_copy(iid_hbm.at[pl.ds(base, _BPW)], iflat_v)

    for j in range(_NCHUNK):
        for k in range(_CHUNK // _LANES):
            sl = pl.ds(k * _LANES, _LANES)
            fl = pl.ds(j * _CHUNK + k * _LANES, _LANES)
            u = uflat_v[fl]
            i = iflat_v[fl]
            uh_v[j, sl] = jnp.where(u < _HU, u, u - _HU)
            ih_v[j, sl] = jnp.where(i < _HI, i, i - _HI)

    for h in range(_NHALF):
        copies = []
        for j in range(_HCHUNK):
            cj = h * _HCHUNK + j
            s = j * _CHUNK
            copies.append(pltpu.async_copy(
                ut_hbm.at[uh_v.at[cj]], urows_v.at[pl.ds(s, _CHUNK)], sem))
            copies.append(pltpu.async_copy(
                it_hbm.at[ih_v.at[cj]], irows_v.at[pl.ds(s, _CHUNK)], sem))
        for c in copies:
            c.wait()

        def group(g, carry):
            fl = pl.ds(h * _HBPW + g * _LANES, _LANES)
            pl_local = g * _LANES + lax.iota(jnp.int32, _LANES)
            uidx = uflat_v[fl]
            iidx = iflat_v[fl]
            ucol0 = jnp.where(uidx < _HU, 0, _D)
            icol0 = jnp.where(iidx < _HI, 0, _D)
            acc = jnp.zeros((_LANES,), jnp.float32)
            for d in range(_D):
                u = plsc.load_gather(urows_v, [pl_local, ucol0 + d])
                v = plsc.load_gather(irows_v, [pl_local, icol0 + d])
                acc = acc + u * v
            dot_v[fl] = acc
            return carry

        lax.fori_loop(0, _NGROUP, group, 0)

    pltpu.sync_copy(dot_v, dot_hbm.at[pl.ds(base, _BPW)])


def _bias_body(uid_hbm, iid_hbm, ub_hbm, ib_hbm, gb_hbm, dot_hbm, out_hbm,
               uidx_v, iidx_v, uq_v, iq_v, ubias_v, ibias_v, dot_v, out_v,
               gb_v, sem):
    wid = lax.axis_index("s") * _NC + lax.axis_index("c")
    cbase = wid * _NCHUNK
    base = wid * _BPW

    pltpu.sync_copy(uid_hbm.at[pl.ds(cbase, _NCHUNK)], uidx_v)
    pltpu.sync_copy(iid_hbm.at[pl.ds(cbase, _NCHUNK)], iidx_v)
    pltpu.sync_copy(dot_hbm.at[pl.ds(base, _BPW)], dot_v)
    pltpu.sync_copy(gb_hbm, gb_v)

    for j in range(_NCHUNK):
        for k in range(_CHUNK // _LANES):
            sl = pl.ds(k * _LANES, _LANES)
            uq_v[j, sl] = uidx_v[j, sl] >> 4
            iq_v[j, sl] = iidx_v[j, sl] >> 4

    copies = []
    for j in range(_NCHUNK):
        s = j * _CHUNK
        copies.append(pltpu.async_copy(
            ub_hbm.at[uq_v.at[j]], ubias_v.at[pl.ds(s, _CHUNK)], sem))
        copies.append(pltpu.async_copy(
            ib_hbm.at[iq_v.at[j]], ibias_v.at[pl.ds(s, _CHUNK)], sem))
    for c in copies:
        c.wait()

    gb = gb_v[...]

    for j in range(_NCHUNK):
        for k in range(_CHUNK // _LANES):
            sl = pl.ds(k * _LANES, _LANES)
            p = j * _CHUNK + k * _LANES + lax.iota(jnp.int32, _LANES)
            uidx = uidx_v[j, sl]
            iidx = iidx_v[j, sl]
            ub = plsc.load_gather(ubias_v, [p, uidx & 15])
            ib = plsc.load_gather(ibias_v, [p, iidx & 15])
            z = dot_v[pl.ds(j * _CHUNK + k * _LANES, _LANES)] + ub + ib + gb
            out_v[pl.ds(j * _CHUNK + k * _LANES, _LANES)] = (
                1.0 / (1.0 + jnp.exp(-z)))

    pltpu.sync_copy(out_v, out_hbm.at[pl.ds(base, _BPW)])


@jax.jit
def _bmf(uid_flat, iid_flat, uid2, iid2, utT, itT, ubq, ibq, gb):
    ut_packed = _pack_halves(utT, _HU)
    it_packed = _pack_halves(itT, _HI)

    mesh = plsc.VectorSubcoreMesh(core_axis_name="c", subcore_axis_name="s")
    emb = pl.kernel(
        _emb_body,
        mesh=mesh,
        compiler_params=pltpu.CompilerParams(
            needs_layout_passes=False, use_tc_tiling_on_sc=True),
        out_type=jax.ShapeDtypeStruct((_B,), jnp.float32),
        scratch_types=[
            pltpu.VMEM((_BPW,), jnp.int32),
            pltpu.VMEM((_BPW,), jnp.int32),
            pltpu.VMEM((_NCHUNK, _CHUNK), jnp.int32),
            pltpu.VMEM((_NCHUNK, _CHUNK), jnp.int32),
            pltpu.VMEM((_HBPW, 2 * _D), jnp.float32),
            pltpu.VMEM((_HBPW, 2 * _D), jnp.float32),
            pltpu.VMEM((_BPW,), jnp.float32),
            pltpu.SemaphoreType.DMA,
        ],
    )
    dots = emb(uid_flat, iid_flat, ut_packed, it_packed)

    bias = pl.kernel(
        _bias_body,
        mesh=mesh,
        compiler_params=pltpu.CompilerParams(
            needs_layout_passes=False, use_tc_tiling_on_sc=False),
        out_type=jax.ShapeDtypeStruct((_B,), jnp.float32),
        scratch_types=[
            pltpu.VMEM((_NCHUNK, _CHUNK), jnp.int32),
            pltpu.VMEM((_NCHUNK, _CHUNK), jnp.int32),
            pltpu.VMEM((_NCHUNK, _CHUNK), jnp.int32),
            pltpu.VMEM((_NCHUNK, _CHUNK), jnp.int32),
            pltpu.VMEM((_BPW, _LANES), jnp.float32),
            pltpu.VMEM((_BPW, _LANES), jnp.float32),
            pltpu.VMEM((_BPW,), jnp.float32),
            pltpu.VMEM((_BPW,), jnp.float32),
            pltpu.VMEM((_LANES,), jnp.float32),
            pltpu.SemaphoreType.DMA,
        ],
    )
    return bias(uid2, iid2, ubq, ibq, gb, dots)


def kernel(user_ids, item_ids, user_table, item_table, user_bias_table,
           item_bias_table, global_bias):
    uid_flat = user_ids.astype(jnp.int32)
    iid_flat = item_ids.astype(jnp.int32)
    uid2 = uid_flat.reshape(_B // _CHUNK, _CHUNK)
    iid2 = iid_flat.reshape(_B // _CHUNK, _CHUNK)
    utT = jnp.transpose(user_table, (1, 0))
    itT = jnp.transpose(item_table, (1, 0))
    ubq = user_bias_table.reshape(-1, _LANES)
    ibq = item_bias_table.reshape(-1, _LANES)
    gb = jnp.broadcast_to(global_bias.reshape(()), (_LANES,))
    out = _bmf(uid_flat, iid_flat, uid2, iid2, utT, itT, ubq, ibq, gb)
    return out.reshape(_B, 1)


# speedup vs baseline: 2.4805x; 2.4805x over previous
import jax
import jax.numpy as jnp
from jax import lax
from jax.experimental import pallas as pl
from jax.experimental.pallas import tpu as pltpu
from jax.experimental.pallas import tpu_sc as plsc

_B = 16384
_D = 64
_LANES = 16
_CHUNK = 128

_NC = 2
_NS = 16
_NW = _NC * _NS
_BPW = _B // _NW
_NCHUNK = _BPW // _CHUNK
_NHALF = 2
_HBPW = _BPW // _NHALF
_HCHUNK = _HBPW // _CHUNK
_NGROUP = _HBPW // _LANES

_PANEL = 4096
_HU = 524288
_HI = 65536


def _tpose_body(a_ref, b_ref, o_ref):
    r = lax.broadcasted_iota(jnp.int32, (_D, _D), 0)
    c = lax.broadcasted_iota(jnp.int32, (_D, _D), 1)
    eye = (r == c).astype(jnp.float32)
    dn = (((0,), (0,)), ((), ()))
    o_ref[:, 0:_D] = lax.dot_general(
        a_ref[...], eye, dn, preferred_element_type=jnp.float32)
    o_ref[:, _D:2 * _D] = lax.dot_general(
        b_ref[...], eye, dn, preferred_element_type=jnp.float32)


def _pack_halves(tT, half):
    grid = half // _PANEL
    n = tT.shape[1]
    last_b = (n - 1) // _PANEL

    def b_map(i):
        return 0, jnp.minimum(i + half // _PANEL, last_b)

    return pl.pallas_call(
        _tpose_body,
        grid=(grid,),
        in_specs=[
            pl.BlockSpec((_D, _PANEL), lambda i: (0, i)),
            pl.BlockSpec((_D, _PANEL), b_map),
        ],
        out_specs=pl.BlockSpec((_PANEL, 2 * _D), lambda i: (i, 0)),
        out_shape=jax.ShapeDtypeStruct((half, 2 * _D), jnp.float32),
    )(tT, tT)


def _emb_body(uid_hbm, iid_hbm, ut_hbm, it_hbm, dot_hbm,
              uflat_v, iflat_v, uh_v, ih_v, urows_v, irows_v, dot_v, sem):
    wid = lax.axis_index("s") * _NC + lax.axis_index("c")
    base = wid * _BPW

    pltpu.sync_copy(uid_hbm.at[pl.ds(base, _BPW)], uflat_v)
    pltpu.sync_copy(iid_hbm.at[pl.ds(base, _BPW)], iflat_v)

    for j in range(_NCHUNK):
        for k in range(_CHUNK // _LANES):
            sl = pl.ds(k * _LANES, _LANES)
            fl = pl.ds(j * _CHUNK + k * _LANES, _LANES)
            u = uflat_v[fl]
            i = iflat_v[fl]
            uh_v[j, sl] = jnp.where(u < _HU, u, u - _HU)
            ih_v[j, sl] = jnp.where(i < _HI, i, i - _HI)

    for h in range(_NHALF):
        copies = []
        for j in range(_HCHUNK):
            cj = h * _HCHUNK + j
            s = j * _CHUNK
            copies.append(pltpu.async_copy(
                ut_hbm.at[uh_v.at[cj]], urows_v.at[pl.ds(s, _CHUNK)], sem))
            copies.append(pltpu.async_copy(
                it_hbm.at[ih_v.at[cj]], irows_v.at[pl.ds(s, _CHUNK)], sem))
        for c in copies:
            c.wait()

        def group(g, carry):
            fl = pl.ds(h * _HBPW + g * _LANES, _LANES)
            pl_local = g * _LANES + lax.iota(jnp.int32, _LANES)
            uidx = uflat_v[fl]
            iidx = iflat_v[fl]
            ucol0 = jnp.where(uidx < _HU, 0, _D)
            icol0 = jnp.where(iidx < _HI, 0, _D)
            acc = jnp.zeros((_LANES,), jnp.float32)
            for d in range(_D):
                u = plsc.load_gather(urows_v, [pl_local, ucol0 + d])
                v = plsc.load_gather(irows_v, [pl_local, icol0 + d])
                acc = acc + u * v
            dot_v[fl] = acc
            return carry

        lax.fori_loop(0, _NGROUP, group, 0)

    pltpu.sync_copy(dot_v, dot_hbm.at[pl.ds(base, _BPW)])


def _bias_body(uid_hbm, iid_hbm, ub_hbm, ib_hbm, gb_hbm, dot_hbm, out_hbm,
               uidx_v, iidx_v, uq_v, iq_v, ubias_v, ibias_v, dot_v, out_v,
               gb_v, sem):
    wid = lax.axis_index("s") * _NC + lax.axis_index("c")
    cbase = wid * _NCHUNK
    base = wid * _BPW

    pltpu.sync_copy(uid_hbm.at[pl.ds(cbase, _NCHUNK)], uidx_v)
    pltpu.sync_copy(iid_hbm.at[pl.ds(cbase, _NCHUNK)], iidx_v)
    pltpu.sync_copy(dot_hbm.at[pl.ds(base, _BPW)], dot_v)
    pltpu.sync_copy(gb_hbm, gb_v)

    for j in range(_NCHUNK):
        for k in range(_CHUNK // _LANES):
            sl = pl.ds(k * _LANES, _LANES)
            uq_v[j, sl] = uidx_v[j, sl] >> 4
            iq_v[j, sl] = iidx_v[j, sl] >> 4

    copies = []
    for j in range(_NCHUNK):
        s = j * _CHUNK
        copies.append(pltpu.async_copy(
            ub_hbm.at[uq_v.at[j]], ubias_v.at[pl.ds(s, _CHUNK)], sem))
        copies.append(pltpu.async_copy(
            ib_hbm.at[iq_v.at[j]], ibias_v.at[pl.ds(s, _CHUNK)], sem))
    for c in copies:
        c.wait()

    gb = gb_v[...]

    for j in range(_NCHUNK):
        for k in range(_CHUNK // _LANES):
            sl = pl.ds(k * _LANES, _LANES)
            p = j * _CHUNK + k * _LANES + lax.iota(jnp.int32, _LANES)
            uidx = uidx_v[j, sl]
            iidx = iidx_v[j, sl]
            ub = plsc.load_gather(ubias_v, [p, uidx & 15])
            ib = plsc.load_gather(ibias_v, [p, iidx & 15])
            z = dot_v[pl.ds(j * _CHUNK + k * _LANES, _LANES)] + ub + ib + gb
            out_v[pl.ds(j * _CHUNK + k * _LANES, _LANES)] = (
                1.0 / (1.0 + jnp.exp(-z)))

    pltpu.sync_copy(out_v, out_hbm.at[pl.ds(base, _BPW)])


@jax.jit
def _bmf(uid_flat, iid_flat, uid2, iid2, utT, itT, ubq, ibq, gb):
    ut_packed = _pack_halves(utT, _HU)
    it_packed = _pack_halves(itT, _HI)

    mesh = plsc.VectorSubcoreMesh(core_axis_name="c", subcore_axis_name="s")
    emb = pl.kernel(
        _emb_body,
        mesh=mesh,
        compiler_params=pltpu.CompilerParams(
            needs_layout_passes=False, use_tc_tiling_on_sc=True),
        out_type=jax.ShapeDtypeStruct((_B,), jnp.float32),
        scratch_types=[
            pltpu.VMEM((_BPW,), jnp.int32),
            pltpu.VMEM((_BPW,), jnp.int32),
            pltpu.VMEM((_NCHUNK, _CHUNK), jnp.int32),
            pltpu.VMEM((_NCHUNK, _CHUNK), jnp.int32),
            pltpu.VMEM((_HBPW, 2 * _D), jnp.float32),
            pltpu.VMEM((_HBPW, 2 * _D), jnp.float32),
            pltpu.VMEM((_BPW,), jnp.float32),
            pltpu.SemaphoreType.DMA,
        ],
    )
    dots = emb(uid_flat, iid_flat, ut_packed, it_packed)

    bias = pl.kernel(
        _bias_body,
        mesh=mesh,
        compiler_params=pltpu.CompilerParams(
            needs_layout_passes=False, use_tc_tiling_on_sc=False),
        out_type=jax.ShapeDtypeStruct((_B,), jnp.float32),
        scratch_types=[
            pltpu.VMEM((_NCHUNK, _CHUNK), jnp.int32),
            pltpu.VMEM((_NCHUNK, _CHUNK), jnp.int32),
            pltpu.VMEM((_NCHUNK, _CHUNK), jnp.int32),
            pltpu.VMEM((_NCHUNK, _CHUNK), jnp.int32),
            pltpu.VMEM((_BPW, _LANES), jnp.float32),
            pltpu.VMEM((_BPW, _LANES), jnp.float32),
            pltpu.VMEM((_BPW,), jnp.float32),
            pltpu.VMEM((_BPW,), jnp.float32),
            pltpu.VMEM((_LANES,), jnp.float32),
            pltpu.SemaphoreType.DMA,
        ],
    )
    return bias(uid2, iid2, ubq, ibq, gb, dots)


def kernel(user_ids, item_ids, user_table, item_table, user_bias_table,
           item_bias_table, global_bias):
    uid_flat = user_ids.astype(jnp.int32)
    iid_flat = item_ids.astype(jnp.int32)
    uid2 = uid_flat.reshape(_B // _CHUNK, _CHUNK)
    iid2 = iid_flat.reshape(_B // _CHUNK, _CHUNK)
    utT = jnp.transpose(user_table, (1, 0))
    itT = jnp.transpose(item_table, (1, 0))
    ubq = user_bias_table.reshape(-1, _LANES)
    ibq = item_bias_table.reshape(-1, _LANES)
    gb = jnp.broadcast_to(global_bias.reshape(()), (_LANES,))
    out = _bmf(uid_flat, iid_flat, uid2, iid2, utT, itT, ubq, ibq, gb)
    return out.reshape(_B, 1)


# speedup vs baseline: 2.7609x; 1.1131x over previous
import jax
import jax.numpy as jnp
from jax import lax
from jax.experimental import pallas as pl
from jax.experimental.pallas import tpu as pltpu
from jax.experimental.pallas import tpu_sc as plsc

_B = 16384
_D = 64
_LANES = 16
_CHUNK = 128

_NC = 2
_NS = 16
_NW = _NC * _NS
_BPW = _B // _NW
_NCHUNK = _BPW // _CHUNK
_NHALF = 2
_HBPW = _BPW // _NHALF
_HCHUNK = _HBPW // _CHUNK
_NGROUP = _HBPW // _LANES

_PANEL = 8192
_HU = 524288
_HI = 65536


def _tpose_body(a_ref, b_ref, o_ref):
    r = lax.broadcasted_iota(jnp.int32, (_D, _D), 0)
    c = lax.broadcasted_iota(jnp.int32, (_D, _D), 1)
    eye = (r == c).astype(jnp.float32)
    dn = (((0,), (0,)), ((), ()))
    o_ref[:, 0:_D] = lax.dot_general(
        a_ref[...], eye, dn, preferred_element_type=jnp.float32)
    o_ref[:, _D:2 * _D] = lax.dot_general(
        b_ref[...], eye, dn, preferred_element_type=jnp.float32)


def _pack_halves(tT, half):
    grid = half // _PANEL
    n = tT.shape[1]
    last_b = (n - 1) // _PANEL

    def b_map(i):
        return 0, jnp.minimum(i + half // _PANEL, last_b)

    return pl.pallas_call(
        _tpose_body,
        grid=(grid,),
        in_specs=[
            pl.BlockSpec((_D, _PANEL), lambda i: (0, i)),
            pl.BlockSpec((_D, _PANEL), b_map),
        ],
        out_specs=pl.BlockSpec((_PANEL, 2 * _D), lambda i: (i, 0)),
        out_shape=jax.ShapeDtypeStruct((half, 2 * _D), jnp.float32),
    )(tT, tT)


def _emb_body(uid_hbm, iid_hbm, ut_hbm, it_hbm, dot_hbm,
              uflat_v, iflat_v, uh_v, ih_v, urows_v, irows_v, dot_v, sem):
    wid = lax.axis_index("s") * _NC + lax.axis_index("c")
    base = wid * _BPW

    pltpu.sync_copy(uid_hbm.at[pl.ds(base, _BPW)], uflat_v)
    pltpu.sync_copy(iid_hbm.at[pl.ds(base, _BPW)], iflat_v)

    for j in range(_NCHUNK):
        for k in range(_CHUNK // _LANES):
            sl = pl.ds(k * _LANES, _LANES)
            fl = pl.ds(j * _CHUNK + k * _LANES, _LANES)
            u = uflat_v[fl]
            i = iflat_v[fl]
            uh_v[j, sl] = jnp.where(u < _HU, u, u - _HU)
            ih_v[j, sl] = jnp.where(i < _HI, i, i - _HI)

    for h in range(_NHALF):
        copies = []
        for j in range(_HCHUNK):
            cj = h * _HCHUNK + j
            s = j * _CHUNK
            copies.append(pltpu.async_copy(
                ut_hbm.at[uh_v.at[cj]], urows_v.at[pl.ds(s, _CHUNK)], sem))
            copies.append(pltpu.async_copy(
                it_hbm.at[ih_v.at[cj]], irows_v.at[pl.ds(s, _CHUNK)], sem))
        for c in copies:
            c.wait()

        def group(g, carry):
            fl = pl.ds(h * _HBPW + g * _LANES, _LANES)
            pl_local = g * _LANES + lax.iota(jnp.int32, _LANES)
            uidx = uflat_v[fl]
            iidx = iflat_v[fl]
            ucol0 = jnp.where(uidx < _HU, 0, _D)
            icol0 = jnp.where(iidx < _HI, 0, _D)
            acc = jnp.zeros((_LANES,), jnp.float32)
            for d in range(_D):
                u = plsc.load_gather(urows_v, [pl_local, ucol0 + d])
                v = plsc.load_gather(irows_v, [pl_local, icol0 + d])
                acc = acc + u * v
            dot_v[fl] = acc
            return carry

        lax.fori_loop(0, _NGROUP, group, 0)

    pltpu.sync_copy(dot_v, dot_hbm.at[pl.ds(base, _BPW)])


def _bias_body(uid_hbm, iid_hbm, ub_hbm, ib_hbm, gb_hbm, dot_hbm, out_hbm,
               uidx_v, iidx_v, uq_v, iq_v, ubias_v, ibias_v, dot_v, out_v,
               gb_v, sem):
    wid = lax.axis_index("s") * _NC + lax.axis_index("c")
    cbase = wid * _NCHUNK
    base = wid * _BPW

    pltpu.sync_copy(uid_hbm.at[pl.ds(cbase, _NCHUNK)], uidx_v)
    pltpu.sync_copy(iid_hbm.at[pl.ds(cbase, _NCHUNK)], iidx_v)
    pltpu.sync_copy(dot_hbm.at[pl.ds(base, _BPW)], dot_v)
    pltpu.sync_copy(gb_hbm, gb_v)

    for j in range(_NCHUNK):
        for k in range(_CHUNK // _LANES):
            sl = pl.ds(k * _LANES, _LANES)
            uq_v[j, sl] = uidx_v[j, sl] >> 4
            iq_v[j, sl] = iidx_v[j, sl] >> 4

    copies = []
    for j in range(_NCHUNK):
        s = j * _CHUNK
        copies.append(pltpu.async_copy(
            ub_hbm.at[uq_v.at[j]], ubias_v.at[pl.ds(s, _CHUNK)], sem))
        copies.append(pltpu.async_copy(
            ib_hbm.at[iq_v.at[j]], ibias_v.at[pl.ds(s, _CHUNK)], sem))
    for c in copies:
        c.wait()

    gb = gb_v[...]

    for j in range(_NCHUNK):
        for k in range(_CHUNK // _LANES):
            sl = pl.ds(k * _LANES, _LANES)
            p = j * _CHUNK + k * _LANES + lax.iota(jnp.int32, _LANES)
            uidx = uidx_v[j, sl]
            iidx = iidx_v[j, sl]
            ub = plsc.load_gather(ubias_v, [p, uidx & 15])
            ib = plsc.load_gather(ibias_v, [p, iidx & 15])
            z = dot_v[pl.ds(j * _CHUNK + k * _LANES, _LANES)] + ub + ib + gb
            out_v[pl.ds(j * _CHUNK + k * _LANES, _LANES)] = (
                1.0 / (1.0 + jnp.exp(-z)))

    pltpu.sync_copy(out_v, out_hbm.at[pl.ds(base, _BPW)])


@jax.jit
def _bmf(uid_flat, iid_flat, uid2, iid2, utT, itT, ubq, ibq, gb):
    ut_packed = _pack_halves(utT, _HU)
    it_packed = _pack_halves(itT, _HI)

    mesh = plsc.VectorSubcoreMesh(core_axis_name="c", subcore_axis_name="s")
    emb = pl.kernel(
        _emb_body,
        mesh=mesh,
        compiler_params=pltpu.CompilerParams(
            needs_layout_passes=False, use_tc_tiling_on_sc=True),
        out_type=jax.ShapeDtypeStruct((_B,), jnp.float32),
        scratch_types=[
            pltpu.VMEM((_BPW,), jnp.int32),
            pltpu.VMEM((_BPW,), jnp.int32),
            pltpu.VMEM((_NCHUNK, _CHUNK), jnp.int32),
            pltpu.VMEM((_NCHUNK, _CHUNK), jnp.int32),
            pltpu.VMEM((_HBPW, 2 * _D), jnp.float32),
            pltpu.VMEM((_HBPW, 2 * _D), jnp.float32),
            pltpu.VMEM((_BPW,), jnp.float32),
            pltpu.SemaphoreType.DMA,
        ],
    )
    dots = emb(uid_flat, iid_flat, ut_packed, it_packed)

    bias = pl.kernel(
        _bias_body,
        mesh=mesh,
        compiler_params=pltpu.CompilerParams(
            needs_layout_passes=False, use_tc_tiling_on_sc=False),
        out_type=jax.ShapeDtypeStruct((_B,), jnp.float32),
        scratch_types=[
            pltpu.VMEM((_NCHUNK, _CHUNK), jnp.int32),
            pltpu.VMEM((_NCHUNK, _CHUNK), jnp.int32),
            pltpu.VMEM((_NCHUNK, _CHUNK), jnp.int32),
            pltpu.VMEM((_NCHUNK, _CHUNK), jnp.int32),
            pltpu.VMEM((_BPW, _LANES), jnp.float32),
            pltpu.VMEM((_BPW, _LANES), jnp.float32),
            pltpu.VMEM((_BPW,), jnp.float32),
            pltpu.VMEM((_BPW,), jnp.float32),
            pltpu.VMEM((_LANES,), jnp.float32),
            pltpu.SemaphoreType.DMA,
        ],
    )
    return bias(uid2, iid2, ubq, ibq, gb, dots)


def kernel(user_ids, item_ids, user_table, item_table, user_bias_table,
           item_bias_table, global_bias):
    uid_flat = user_ids.astype(jnp.int32)
    iid_flat = item_ids.astype(jnp.int32)
    uid2 = uid_flat.reshape(_B // _CHUNK, _CHUNK)
    iid2 = iid_flat.reshape(_B // _CHUNK, _CHUNK)
    utT = jnp.transpose(user_table, (1, 0))
    itT = jnp.transpose(item_table, (1, 0))
    ubq = user_bias_table.reshape(-1, _LANES)
    ibq = item_bias_table.reshape(-1, _LANES)
    gb = jnp.broadcast_to(global_bias.reshape(()), (_LANES,))
    out = _bmf(uid_flat, iid_flat, uid2, iid2, utT, itT, ubq, ibq, gb)
    return out.reshape(_B, 1)


# speedup vs baseline: 2.8654x; 1.0379x over previous
import jax
import jax.numpy as jnp
from jax import lax
from jax.experimental import pallas as pl
from jax.experimental.pallas import tpu as pltpu
from jax.experimental.pallas import tpu_sc as plsc

_B = 16384
_D = 64
_LANES = 16
_CHUNK = 128

_NC = 2
_NS = 16
_NW = _NC * _NS
_BPW = _B // _NW
_NCHUNK = _BPW // _CHUNK
_NHALF = 2
_HBPW = _BPW // _NHALF
_HCHUNK = _HBPW // _CHUNK
_NGROUP = _HBPW // _LANES

_PANEL = 16384
_HU = 524288
_HI = 65536


def _tpose_body(a_ref, b_ref, o_ref):
    r = lax.broadcasted_iota(jnp.int32, (_D, _D), 0)
    c = lax.broadcasted_iota(jnp.int32, (_D, _D), 1)
    eye = (r == c).astype(jnp.float32)
    dn = (((0,), (0,)), ((), ()))
    o_ref[:, 0:_D] = lax.dot_general(
        a_ref[...], eye, dn, preferred_element_type=jnp.float32)
    o_ref[:, _D:2 * _D] = lax.dot_general(
        b_ref[...], eye, dn, preferred_element_type=jnp.float32)


def _pack_halves(tT, half):
    grid = half // _PANEL
    n = tT.shape[1]
    last_b = (n - 1) // _PANEL

    def b_map(i):
        return 0, jnp.minimum(i + half // _PANEL, last_b)

    return pl.pallas_call(
        _tpose_body,
        grid=(grid,),
        in_specs=[
            pl.BlockSpec((_D, _PANEL), lambda i: (0, i)),
            pl.BlockSpec((_D, _PANEL), b_map),
        ],
        out_specs=pl.BlockSpec((_PANEL, 2 * _D), lambda i: (i, 0)),
        out_shape=jax.ShapeDtypeStruct((half, 2 * _D), jnp.float32),
    )(tT, tT)


def _emb_body(uid_hbm, iid_hbm, ut_hbm, it_hbm, dot_hbm,
              uflat_v, iflat_v, uh_v, ih_v, urows_v, irows_v, dot_v, sem):
    wid = lax.axis_index("s") * _NC + lax.axis_index("c")
    base = wid * _BPW

    pltpu.sync_copy(uid_hbm.at[pl.ds(base, _BPW)], uflat_v)
    pltpu.sync_copy(iid_hbm.at[pl.ds(base, _BPW)], iflat_v)

    for j in range(_NCHUNK):
        for k in range(_CHUNK // _LANES):
            sl = pl.ds(k * _LANES, _LANES)
            fl = pl.ds(j * _CHUNK + k * _LANES, _LANES)
            u = uflat_v[fl]
            i = iflat_v[fl]
            uh_v[j, sl] = jnp.where(u < _HU, u, u - _HU)
            ih_v[j, sl] = jnp.where(i < _HI, i, i - _HI)

    for h in range(_NHALF):
        copies = []
        for j in range(_HCHUNK):
            cj = h * _HCHUNK + j
            s = j * _CHUNK
            copies.append(pltpu.async_copy(
                ut_hbm.at[uh_v.at[cj]], urows_v.at[pl.ds(s, _CHUNK)], sem))
            copies.append(pltpu.async_copy(
                it_hbm.at[ih_v.at[cj]], irows_v.at[pl.ds(s, _CHUNK)], sem))
        for c in copies:
            c.wait()

        def group(g, carry):
            fl = pl.ds(h * _HBPW + g * _LANES, _LANES)
            pl_local = g * _LANES + lax.iota(jnp.int32, _LANES)
            uidx = uflat_v[fl]
            iidx = iflat_v[fl]
            ucol0 = jnp.where(uidx < _HU, 0, _D)
            icol0 = jnp.where(iidx < _HI, 0, _D)
            acc = jnp.zeros((_LANES,), jnp.float32)
            for d in range(_D):
                u = plsc.load_gather(urows_v, [pl_local, ucol0 + d])
                v = plsc.load_gather(irows_v, [pl_local, icol0 + d])
                acc = acc + u * v
            dot_v[fl] = acc
            return carry

        lax.fori_loop(0, _NGROUP, group, 0)

    pltpu.sync_copy(dot_v, dot_hbm.at[pl.ds(base, _BPW)])


def _bias_body(uid_hbm, iid_hbm, ub_hbm, ib_hbm, gb_hbm, dot_hbm, out_hbm,
               uidx_v, iidx_v, uq_v, iq_v, ubias_v, ibias_v, dot_v, out_v,
               gb_v, sem):
    wid = lax.axis_index("s") * _NC + lax.axis_index("c")
    cbase = wid * _NCHUNK
    base = wid * _BPW

    pltpu.sync_copy(uid_hbm.at[pl.ds(cbase, _NCHUNK)], uidx_v)
    pltpu.sync_copy(iid_hbm.at[pl.ds(cbase, _NCHUNK)], iidx_v)
    pltpu.sync_copy(dot_hbm.at[pl.ds(base, _BPW)], dot_v)
    pltpu.sync_copy(gb_hbm, gb_v)

    for j in range(_NCHUNK):
        for k in range(_CHUNK // _LANES):
            sl = pl.ds(k * _LANES, _LANES)
            uq_v[j, sl] = uidx_v[j, sl] >> 4
            iq_v[j, sl] = iidx_v[j, sl] >> 4

    copies = []
    for j in range(_NCHUNK):
        s = j * _CHUNK
        copies.append(pltpu.async_copy(
            ub_hbm.at[uq_v.at[j]], ubias_v.at[pl.ds(s, _CHUNK)], sem))
        copies.append(pltpu.async_copy(
            ib_hbm.at[iq_v.at[j]], ibias_v.at[pl.ds(s, _CHUNK)], sem))
    for c in copies:
        c.wait()

    gb = gb_v[...]

    for j in range(_NCHUNK):
        for k in range(_CHUNK // _LANES):
            sl = pl.ds(k * _LANES, _LANES)
            p = j * _CHUNK + k * _LANES + lax.iota(jnp.int32, _LANES)
            uidx = uidx_v[j, sl]
            iidx = iidx_v[j, sl]
            ub = plsc.load_gather(ubias_v, [p, uidx & 15])
            ib = plsc.load_gather(ibias_v, [p, iidx & 15])
            z = dot_v[pl.ds(j * _CHUNK + k * _LANES, _LANES)] + ub + ib + gb
            out_v[pl.ds(j * _CHUNK + k * _LANES, _LANES)] = (
                1.0 / (1.0 + jnp.exp(-z)))

    pltpu.sync_copy(out_v, out_hbm.at[pl.ds(base, _BPW)])


@jax.jit
def _bmf(uid_flat, iid_flat, uid2, iid2, utT, itT, ubq, ibq, gb):
    ut_packed = _pack_halves(utT, _HU)
    it_packed = _pack_halves(itT, _HI)

    mesh = plsc.VectorSubcoreMesh(core_axis_name="c", subcore_axis_name="s")
    emb = pl.kernel(
        _emb_body,
        mesh=mesh,
        compiler_params=pltpu.CompilerParams(
            needs_layout_passes=False, use_tc_tiling_on_sc=True),
        out_type=jax.ShapeDtypeStruct((_B,), jnp.float32),
        scratch_types=[
            pltpu.VMEM((_BPW,), jnp.int32),
            pltpu.VMEM((_BPW,), jnp.int32),
            pltpu.VMEM((_NCHUNK, _CHUNK), jnp.int32),
            pltpu.VMEM((_NCHUNK, _CHUNK), jnp.int32),
            pltpu.VMEM((_HBPW, 2 * _D), jnp.float32),
            pltpu.VMEM((_HBPW, 2 * _D), jnp.float32),
            pltpu.VMEM((_BPW,), jnp.float32),
            pltpu.SemaphoreType.DMA,
        ],
    )
    dots = emb(uid_flat, iid_flat, ut_packed, it_packed)

    bias = pl.kernel(
        _bias_body,
        mesh=mesh,
        compiler_params=pltpu.CompilerParams(
            needs_layout_passes=False, use_tc_tiling_on_sc=False),
        out_type=jax.ShapeDtypeStruct((_B,), jnp.float32),
        scratch_types=[
            pltpu.VMEM((_NCHUNK, _CHUNK), jnp.int32),
            pltpu.VMEM((_NCHUNK, _CHUNK), jnp.int32),
            pltpu.VMEM((_NCHUNK, _CHUNK), jnp.int32),
            pltpu.VMEM((_NCHUNK, _CHUNK), jnp.int32),
            pltpu.VMEM((_BPW, _LANES), jnp.float32),
            pltpu.VMEM((_BPW, _LANES), jnp.float32),
            pltpu.VMEM((_BPW,), jnp.float32),
            pltpu.VMEM((_BPW,), jnp.float32),
            pltpu.VMEM((_LANES,), jnp.float32),
            pltpu.SemaphoreType.DMA,
        ],
    )
    return bias(uid2, iid2, ubq, ibq, gb, dots)


def kernel(user_ids, item_ids, user_table, item_table, user_bias_table,
           item_bias_table, global_bias):
    uid_flat = user_ids.astype(jnp.int32)
    iid_flat = item_ids.astype(jnp.int32)
    uid2 = uid_flat.reshape(_B // _CHUNK, _CHUNK)
    iid2 = iid_flat.reshape(_B // _CHUNK, _CHUNK)
    utT = jnp.transpose(user_table, (1, 0))
    itT = jnp.transpose(item_table, (1, 0))
    ubq = user_bias_table.reshape(-1, _LANES)
    ibq = item_bias_table.reshape(-1, _LANES)
    gb = jnp.broadcast_to(global_bias.reshape(()), (_LANES,))
    out = _bmf(uid_flat, iid_flat, uid2, iid2, utT, itT, ubq, ibq, gb)
    return out.reshape(_B, 1)
